# Initial kernel scaffold; baseline (speedup 1.0000x reference)
import functools

import jax
import jax.numpy as jnp
from jax import lax
from jax.experimental import pallas as pl
from jax.experimental.pallas import tpu as pltpu
from jax.experimental.pallas import tpu_sc as plsc

_N = 10000
_E = 320000
_D_IN = 128
_D_HID = 256
_N_CLS = 64

_NC = 2
_NS = 16
_NW = _NC * _NS
_EPT = _E // _NW
_CH = 80
_NCHUNK = _EPT // _CH
_RPT = _N // _NS
_ZR = 125


def _make_sc_agg(D, with_deg):
  mesh = plsc.VectorSubcoreMesh(core_axis_name="c", subcore_axis_name="s")
  out_type = [jax.ShapeDtypeStruct((_NC, _N, D), jnp.float32)]
  scratch = [
      pltpu.VMEM((_NCHUNK, _CH), jnp.int32),
      pltpu.VMEM((_NCHUNK, _CH), jnp.int32),
      pltpu.VMEM((_CH, D), jnp.float32),
      pltpu.VMEM((_ZR, D), jnp.float32),
      pltpu.VMEM_SHARED((_N, D), jnp.float32),
  ]
  if with_deg:
    out_type.append(jax.ShapeDtypeStruct((_NC, _N, 16), jnp.float32))
    scratch += [
        pltpu.VMEM((_CH, 16), jnp.float32),
        pltpu.VMEM((_ZR, 16), jnp.float32),
        pltpu.VMEM_SHARED((_N, 16), jnp.float32),
    ]

  def body(table_hbm, src_hbm, dst_hbm, *refs):
    if with_deg:
      (agg_out, deg_out, srcv, dstv, rows, zbuf, acc, ones, z16, dacc) = refs
    else:
      (agg_out, srcv, dstv, rows, zbuf, acc) = refs
    c = lax.axis_index("c")
    s = lax.axis_index("s")
    wid = s * _NC + c

    base = wid * _NCHUNK
    pltpu.sync_copy(src_hbm.at[pl.ds(base, _NCHUNK)], srcv)
    pltpu.sync_copy(dst_hbm.at[pl.ds(base, _NCHUNK)], dstv)

    zv = jnp.zeros((16,), jnp.float32)

    def zrow(i, _):
      for j in range(D // 16):
        zbuf[i, pl.ds(j * 16, 16)] = zv
      if with_deg:
        z16[i, pl.ds(0, 16)] = zv
      return 0

    lax.fori_loop(0, _ZR, zrow, 0)
    if with_deg:
      ov = jnp.ones((16,), jnp.float32)

      def orow(i, _):
        ones[i, pl.ds(0, 16)] = ov
        return 0

      lax.fori_loop(0, _CH, orow, 0)

    for k in range(_RPT // _ZR):
      pltpu.sync_copy(zbuf, acc.at[pl.ds(s * _RPT + k * _ZR, _ZR)])
      if with_deg:
        pltpu.sync_copy(z16, dacc.at[pl.ds(s * _RPT + k * _ZR, _ZR)])
    plsc.subcore_barrier()

    def step(ci, _):
      pltpu.sync_copy(table_hbm.at[srcv.at[ci]], rows)
      pltpu.sync_copy(rows, acc.at[dstv.at[ci]], add=True)
      if with_deg:
        pltpu.sync_copy(ones, dacc.at[dstv.at[ci]], add=True)
      return 0

    lax.fori_loop(0, _NCHUNK, step, 0)
    plsc.subcore_barrier()

    pltpu.sync_copy(acc.at[pl.ds(s * _RPT, _RPT)],
                    agg_out.at[c].at[pl.ds(s * _RPT, _RPT)])
    if with_deg:
      pltpu.sync_copy(dacc.at[pl.ds(s * _RPT, _RPT)],
                      deg_out.at[c].at[pl.ds(s * _RPT, _RPT)])

  return pl.kernel(body, out_type=out_type, mesh=mesh, scratch_types=scratch)


_sc_agg128 = _make_sc_agg(_D_IN, with_deg=True)
_sc_agg64 = _make_sc_agg(_N_CLS, with_deg=False)

_BR = 400


def _tc1_body(x_ref, aggp_ref, degp_ref, w1s_ref, w1n_ref, b1_ref,
              w2s_ref, w2n_ref, hs_ref, hw_ref):
  agg = aggp_ref[0] + aggp_ref[1]
  deg16 = degp_ref[0] + degp_ref[1]
  deg = jnp.max(deg16, axis=1, keepdims=True)
  hn = agg / jnp.maximum(deg, 1.0)
  h = (jnp.dot(x_ref[...], w1s_ref[...], preferred_element_type=jnp.float32)
       + jnp.dot(hn, w1n_ref[...], preferred_element_type=jnp.float32)
       + b1_ref[...])
  h = jnp.maximum(h, 0.0)
  hs_ref[...] = jnp.dot(h, w2s_ref[...], preferred_element_type=jnp.float32)
  hw_ref[...] = jnp.dot(h, w2n_ref[...], preferred_element_type=jnp.float32)


_tc1 = pl.pallas_call(
    _tc1_body,
    grid=(_N // _BR,),
    in_specs=[
        pl.BlockSpec((_BR, _D_IN), lambda i: (i, 0)),
        pl.BlockSpec((_NC, _BR, _D_IN), lambda i: (0, i, 0)),
        pl.BlockSpec((_NC, _BR, 16), lambda i: (0, i, 0)),
        pl.BlockSpec((_D_IN, _D_HID), lambda i: (0, 0)),
        pl.BlockSpec((_D_IN, _D_HID), lambda i: (0, 0)),
        pl.BlockSpec((1, _D_HID), lambda i: (0, 0)),
        pl.BlockSpec((_D_HID, _N_CLS), lambda i: (0, 0)),
        pl.BlockSpec((_D_HID, _N_CLS), lambda i: (0, 0)),
    ],
    out_specs=[
        pl.BlockSpec((_BR, _N_CLS), lambda i: (i, 0)),
        pl.BlockSpec((_BR, _N_CLS), lambda i: (i, 0)),
    ],
    out_shape=[
        jax.ShapeDtypeStruct((_N, _N_CLS), jnp.float32),
        jax.ShapeDtypeStruct((_N, _N_CLS), jnp.float32),
    ],
)


def _tc2_body(hs_ref, agg2p_ref, degp_ref, b2_ref, out_ref):
  agg = agg2p_ref[0] + agg2p_ref[1]
  deg16 = degp_ref[0] + degp_ref[1]
  deg = jnp.maximum(jnp.max(deg16, axis=1, keepdims=True), 1.0)
  logits = hs_ref[...] + agg / deg + b2_ref[...]
  m = jnp.max(logits, axis=1, keepdims=True)
  lse = jnp.log(jnp.sum(jnp.exp(logits - m), axis=1, keepdims=True)) + m
  out_ref[...] = logits - lse


_tc2 = pl.pallas_call(
    _tc2_body,
    grid=(_N // _BR,),
    in_specs=[
        pl.BlockSpec((_BR, _N_CLS), lambda i: (i, 0)),
        pl.BlockSpec((_NC, _BR, _N_CLS), lambda i: (0, i, 0)),
        pl.BlockSpec((_NC, _BR, 16), lambda i: (0, i, 0)),
        pl.BlockSpec((1, _N_CLS), lambda i: (0, 0)),
    ],
    out_specs=pl.BlockSpec((_BR, _N_CLS), lambda i: (i, 0)),
    out_shape=jax.ShapeDtypeStruct((_N, _N_CLS), jnp.float32),
)


@jax.jit
def kernel(features, adj, W1_self, W1_neigh, b1, W2_self, W2_neigh, b2):
  src2d = adj[0].reshape(_E // _CH, _CH)
  dst2d = adj[1].reshape(_E // _CH, _CH)
  aggp, degp = _sc_agg128(features, src2d, dst2d)
  hs, hw = _tc1(features, aggp, degp, W1_self, W1_neigh,
                b1.reshape(1, _D_HID), W2_self, W2_neigh)
  (agg2p,) = _sc_agg64(hw, src2d, dst2d)
  return _tc2(hs, agg2p, degp, b2.reshape(1, _N_CLS))


# speedup vs baseline: 8.7770x; 8.7770x over previous
import functools

import jax
import jax.numpy as jnp
from jax import lax
from jax.experimental import pallas as pl
from jax.experimental.pallas import tpu as pltpu
from jax.experimental.pallas import tpu_sc as plsc

_N = 10000
_E = 320000
_D_IN = 128
_D_HID = 256
_N_CLS = 64

_NC = 2
_NS = 16
_NW = _NC * _NS
_EPT = _E // _NW
_CH = 80
_NCHUNK = _EPT // _CH
_RPT = _N // _NS
_ZR = 125


def _make_sc_agg(D, with_deg):
  mesh = plsc.VectorSubcoreMesh(core_axis_name="c", subcore_axis_name="s")
  out_type = [jax.ShapeDtypeStruct((_NC, _NS, _RPT, D), jnp.float32)]
  scratch = [
      pltpu.VMEM((_NCHUNK, _CH), jnp.int32),
      pltpu.VMEM((_NCHUNK, _CH), jnp.int32),
      pltpu.VMEM((_CH, D), jnp.float32),
      pltpu.VMEM_SHARED((_N, D), jnp.float32),
  ]
  if with_deg:
    out_type.append(jax.ShapeDtypeStruct((_NC, _NS, _RPT, 16), jnp.float32))
    scratch += [
        pltpu.VMEM((_CH, 16), jnp.float32),
        pltpu.VMEM((_ZR, 16), jnp.float32),
        pltpu.VMEM_SHARED((_N, 16), jnp.float32),
    ]

  def body(table_hbm, src_hbm, dst_hbm, *refs):
    if with_deg:
      (agg_out, deg_out, srcv, dstv, rows, acc, ones, z16, dacc) = refs
    else:
      (agg_out, srcv, dstv, rows, acc) = refs
    c = lax.axis_index("c")
    s = lax.axis_index("s")
    wid = s * _NC + c

    pltpu.sync_copy(src_hbm.at[wid], srcv)
    pltpu.sync_copy(dst_hbm.at[wid], dstv)

    zv = jnp.zeros((16,), jnp.float32)

    def zrow(i, _):
      for j in range(D // 16):
        rows[i, pl.ds(j * 16, 16)] = zv
      if with_deg:
        ones[i, pl.ds(0, 16)] = jnp.ones((16,), jnp.float32)
      return 0

    lax.fori_loop(0, _CH, zrow, 0)
    if with_deg:

      def z16row(i, _):
        z16[i, pl.ds(0, 16)] = zv
        return 0

      lax.fori_loop(0, _ZR, z16row, 0)

    for k in range(_RPT // _CH):
      pltpu.sync_copy(rows, acc.at[pl.ds(s * _RPT + k * _CH, _CH)])
    rem = _RPT - (_RPT // _CH) * _CH
    if rem:
      pltpu.sync_copy(rows.at[pl.ds(0, rem)],
                      acc.at[pl.ds(s * _RPT + (_RPT // _CH) * _CH, rem)])
    if with_deg:
      for k in range(_RPT // _ZR):
        pltpu.sync_copy(z16, dacc.at[pl.ds(s * _RPT + k * _ZR, _ZR)])
    plsc.subcore_barrier()

    def step(ci, _):
      pltpu.sync_copy(table_hbm.at[srcv.at[ci]], rows)
      pltpu.sync_copy(rows, acc.at[dstv.at[ci]], add=True)
      if with_deg:
        pltpu.sync_copy(ones, dacc.at[dstv.at[ci]], add=True)
      return 0

    lax.fori_loop(0, _NCHUNK, step, 0)
    plsc.subcore_barrier()

    pltpu.sync_copy(acc.at[pl.ds(s * _RPT, _RPT)], agg_out.at[c, s])
    if with_deg:
      pltpu.sync_copy(dacc.at[pl.ds(s * _RPT, _RPT)], deg_out.at[c, s])

  return pl.kernel(
      body, out_type=out_type, mesh=mesh, scratch_types=scratch,
      compiler_params=pltpu.CompilerParams(use_tc_tiling_on_sc=False))


_sc_agg128 = _make_sc_agg(_D_IN, with_deg=True)
_sc_agg64 = _make_sc_agg(_N_CLS, with_deg=False)

_BR = 400


def _tc1_body(x_ref, aggp_ref, degp_ref, w1s_ref, w1n_ref, b1_ref,
              w2s_ref, w2n_ref, hs_ref, hw_ref):
  agg = aggp_ref[0] + aggp_ref[1]
  deg16 = degp_ref[0] + degp_ref[1]
  deg = jnp.max(deg16, axis=1, keepdims=True)
  hn = agg / jnp.maximum(deg, 1.0)
  h = (jnp.dot(x_ref[...], w1s_ref[...], preferred_element_type=jnp.float32)
       + jnp.dot(hn, w1n_ref[...], preferred_element_type=jnp.float32)
       + b1_ref[...])
  h = jnp.maximum(h, 0.0)
  hs_ref[...] = jnp.dot(h, w2s_ref[...], preferred_element_type=jnp.float32)
  hw_ref[...] = jnp.dot(h, w2n_ref[...], preferred_element_type=jnp.float32)


_tc1 = pl.pallas_call(
    _tc1_body,
    grid=(_N // _BR,),
    in_specs=[
        pl.BlockSpec((_BR, _D_IN), lambda i: (i, 0)),
        pl.BlockSpec((_NC, _BR, _D_IN), lambda i: (0, i, 0)),
        pl.BlockSpec((_NC, _BR, 16), lambda i: (0, i, 0)),
        pl.BlockSpec((_D_IN, _D_HID), lambda i: (0, 0)),
        pl.BlockSpec((_D_IN, _D_HID), lambda i: (0, 0)),
        pl.BlockSpec((1, _D_HID), lambda i: (0, 0)),
        pl.BlockSpec((_D_HID, _N_CLS), lambda i: (0, 0)),
        pl.BlockSpec((_D_HID, _N_CLS), lambda i: (0, 0)),
    ],
    out_specs=[
        pl.BlockSpec((_BR, _N_CLS), lambda i: (i, 0)),
        pl.BlockSpec((_BR, _N_CLS), lambda i: (i, 0)),
    ],
    out_shape=[
        jax.ShapeDtypeStruct((_N, _N_CLS), jnp.float32),
        jax.ShapeDtypeStruct((_N, _N_CLS), jnp.float32),
    ],
)


def _tc2_body(hs_ref, agg2p_ref, degp_ref, b2_ref, out_ref):
  agg = agg2p_ref[0] + agg2p_ref[1]
  deg16 = degp_ref[0] + degp_ref[1]
  deg = jnp.maximum(jnp.max(deg16, axis=1, keepdims=True), 1.0)
  logits = hs_ref[...] + agg / deg + b2_ref[...]
  m = jnp.max(logits, axis=1, keepdims=True)
  lse = jnp.log(jnp.sum(jnp.exp(logits - m), axis=1, keepdims=True)) + m
  out_ref[...] = logits - lse


_tc2 = pl.pallas_call(
    _tc2_body,
    grid=(_N // _BR,),
    in_specs=[
        pl.BlockSpec((_BR, _N_CLS), lambda i: (i, 0)),
        pl.BlockSpec((_NC, _BR, _N_CLS), lambda i: (0, i, 0)),
        pl.BlockSpec((_NC, _BR, 16), lambda i: (0, i, 0)),
        pl.BlockSpec((1, _N_CLS), lambda i: (0, 0)),
    ],
    out_specs=pl.BlockSpec((_BR, _N_CLS), lambda i: (i, 0)),
    out_shape=jax.ShapeDtypeStruct((_N, _N_CLS), jnp.float32),
)


@jax.jit
def kernel(features, adj, W1_self, W1_neigh, b1, W2_self, W2_neigh, b2):
  src2d = adj[0].reshape(_NW, _NCHUNK, _CH)
  dst2d = adj[1].reshape(_NW, _NCHUNK, _CH)
  aggp, degp = _sc_agg128(features, src2d, dst2d)
  aggp = aggp.reshape(_NC, _N, _D_IN)
  degp = degp.reshape(_NC, _N, 16)
  hs, hw = _tc1(features, aggp, degp, W1_self, W1_neigh,
                b1.reshape(1, _D_HID), W2_self, W2_neigh)
  (agg2p,) = _sc_agg64(hw, src2d, dst2d)
  agg2p = agg2p.reshape(_NC, _N, _N_CLS)
  return _tc2(hs, agg2p, degp, b2.reshape(1, _N_CLS))


# speedup vs baseline: 10.0389x; 1.1438x over previous
import functools

import jax
import jax.numpy as jnp
from jax import lax
from jax.experimental import pallas as pl
from jax.experimental.pallas import tpu as pltpu
from jax.experimental.pallas import tpu_sc as plsc

_N = 10000
_E = 320000
_D_IN = 128
_D_HID = 256
_N_CLS = 64
_DX = _D_IN + 16

_NC = 2
_NS = 16
_NW = _NC * _NS
_EPT = _E // _NW
_CH = 80
_NCHUNK = _EPT // _CH
_NPAIR = _NCHUNK // 2
_RPT = _N // _NS


def _make_sc_agg(D):
  mesh = plsc.VectorSubcoreMesh(core_axis_name="c", subcore_axis_name="s")
  out_type = jax.ShapeDtypeStruct((_NC, _NS, _RPT, D), jnp.float32)
  scratch = [
      pltpu.VMEM((4, _CH), jnp.int32),
      pltpu.VMEM((4, _CH), jnp.int32),
      pltpu.VMEM((2, _CH, D), jnp.float32),
      pltpu.VMEM_SHARED((_N, D), jnp.float32),
      pltpu.SemaphoreType.DMA,
      pltpu.SemaphoreType.DMA,
      pltpu.SemaphoreType.DMA,
      pltpu.SemaphoreType.DMA,
      pltpu.SemaphoreType.DMA,
  ]

  def body(table, src_hbm, dst_hbm, agg_out,
           srcv, dstv, rows, acc, isem0, isem1, gsem0, gsem1, ssem):
    cc = lax.axis_index("c")
    s = lax.axis_index("s")
    wid = s * _NC + cc

    def islot(c):
      return lax.rem(c, 4)

    def i_issue(c, sem):
      pltpu.async_copy(src_hbm.at[wid, c], srcv.at[islot(c)], sem)
      pltpu.async_copy(dst_hbm.at[wid, c], dstv.at[islot(c)], sem)

    def i_wait(c, sem):
      pltpu.make_async_copy(src_hbm.at[wid, c], srcv.at[islot(c)], sem).wait()
      pltpu.make_async_copy(dst_hbm.at[wid, c], dstv.at[islot(c)], sem).wait()

    def g_issue(c, b, sem):
      pltpu.async_copy(table.at[srcv.at[islot(c)]], rows.at[b], sem)

    def g_wait(c, b, sem):
      pltpu.make_async_copy(table.at[srcv.at[islot(c)]], rows.at[b],
                            sem).wait()

    def s_issue(c, b):
      pltpu.async_copy(rows.at[b], acc.at[dstv.at[islot(c)]], ssem, add=True)

    def s_wait(c, b):
      pltpu.make_async_copy(rows.at[b], acc.at[dstv.at[islot(c)]],
                            ssem).wait()

    i_issue(0, isem0)
    i_issue(1, isem1)

    zv = jnp.zeros((16,), jnp.float32)

    def zrow(i, _):
      for j in range(D // 16):
        rows[0, i, pl.ds(j * 16, 16)] = zv
      return 0

    lax.fori_loop(0, _CH, zrow, 0)
    for k in range(_RPT // _CH):
      pltpu.sync_copy(rows.at[0], acc.at[pl.ds(s * _RPT + k * _CH, _CH)])
    rem = _RPT - (_RPT // _CH) * _CH
    if rem:
      pltpu.sync_copy(rows.at[0, pl.ds(0, rem)],
                      acc.at[pl.ds(s * _RPT + (_RPT // _CH) * _CH, rem)])
    plsc.subcore_barrier()

    i_wait(0, isem0)
    i_issue(2, isem0)
    g_issue(0, 0, gsem0)

    def pair(p, _):
      ce = 2 * p
      co = 2 * p + 1
      g_wait(ce, 0, gsem0)

      @pl.when(p > 0)
      def _():
        s_wait(ce - 1, 1)

      s_issue(ce, 0)
      i_wait(ce + 1, isem1)

      @pl.when(ce + 3 < _NCHUNK)
      def _():
        i_issue(ce + 3, isem1)

      g_issue(ce + 1, 1, gsem1)
      g_wait(co, 1, gsem1)
      s_wait(co - 1, 0)
      s_issue(co, 1)
      i_wait(co + 1, isem0)

      @pl.when(co + 3 < _NCHUNK)
      def _():
        i_issue(co + 3, isem0)

      g_issue(co + 1, 0, gsem0)
      return 0

    lax.fori_loop(0, _NPAIR, pair, 0)

    ct = _NCHUNK - 1
    g_wait(ct, 0, gsem0)
    s_wait(ct - 1, 1)
    s_issue(ct, 0)
    s_wait(ct, 0)
    plsc.subcore_barrier()

    pltpu.sync_copy(acc.at[pl.ds(s * _RPT, _RPT)], agg_out.at[cc, s])

  return pl.kernel(
      body, out_type=out_type, mesh=mesh, scratch_types=scratch,
      compiler_params=pltpu.CompilerParams(use_tc_tiling_on_sc=False))


_sc_agg_x = _make_sc_agg(_DX)
_sc_agg_h = _make_sc_agg(_N_CLS)

_BR = 400


def _tc1_body(x_ref, aggxp_ref, w1s_ref, w1n_ref, b1_ref,
              w2s_ref, w2n_ref, hs_ref, hw_ref, rdeg_ref):
  ap = aggxp_ref[0] + aggxp_ref[1]
  agg = ap[:, :_D_IN]
  deg = jnp.max(ap[:, _D_IN:], axis=1, keepdims=True)
  rdeg = 1.0 / jnp.maximum(deg, 1.0)
  hn = agg * rdeg
  h = (jnp.dot(x_ref[...], w1s_ref[...], preferred_element_type=jnp.float32)
       + jnp.dot(hn, w1n_ref[...], preferred_element_type=jnp.float32)
       + b1_ref[...])
  h = jnp.maximum(h, 0.0)
  hs_ref[...] = jnp.dot(h, w2s_ref[...], preferred_element_type=jnp.float32)
  hw_ref[...] = jnp.dot(h, w2n_ref[...], preferred_element_type=jnp.float32)
  rdeg_ref[...] = jnp.broadcast_to(rdeg, (_BR, _N_CLS))


_tc1 = pl.pallas_call(
    _tc1_body,
    grid=(_N // _BR,),
    in_specs=[
        pl.BlockSpec((_BR, _D_IN), lambda i: (i, 0)),
        pl.BlockSpec((_NC, _BR, _DX), lambda i: (0, i, 0)),
        pl.BlockSpec((_D_IN, _D_HID), lambda i: (0, 0)),
        pl.BlockSpec((_D_IN, _D_HID), lambda i: (0, 0)),
        pl.BlockSpec((1, _D_HID), lambda i: (0, 0)),
        pl.BlockSpec((_D_HID, _N_CLS), lambda i: (0, 0)),
        pl.BlockSpec((_D_HID, _N_CLS), lambda i: (0, 0)),
    ],
    out_specs=[
        pl.BlockSpec((_BR, _N_CLS), lambda i: (i, 0)),
        pl.BlockSpec((_BR, _N_CLS), lambda i: (i, 0)),
        pl.BlockSpec((_BR, _N_CLS), lambda i: (i, 0)),
    ],
    out_shape=[
        jax.ShapeDtypeStruct((_N, _N_CLS), jnp.float32),
        jax.ShapeDtypeStruct((_N, _N_CLS), jnp.float32),
        jax.ShapeDtypeStruct((_N, _N_CLS), jnp.float32),
    ],
)


def _tc2_body(hs_ref, agg2p_ref, rdeg_ref, b2_ref, out_ref):
  agg = agg2p_ref[0] + agg2p_ref[1]
  logits = hs_ref[...] + agg * rdeg_ref[...] + b2_ref[...]
  m = jnp.max(logits, axis=1, keepdims=True)
  lse = jnp.log(jnp.sum(jnp.exp(logits - m), axis=1, keepdims=True)) + m
  out_ref[...] = logits - lse


_tc2 = pl.pallas_call(
    _tc2_body,
    grid=(_N // _BR,),
    in_specs=[
        pl.BlockSpec((_BR, _N_CLS), lambda i: (i, 0)),
        pl.BlockSpec((_NC, _BR, _N_CLS), lambda i: (0, i, 0)),
        pl.BlockSpec((_BR, _N_CLS), lambda i: (i, 0)),
        pl.BlockSpec((1, _N_CLS), lambda i: (0, 0)),
    ],
    out_specs=pl.BlockSpec((_BR, _N_CLS), lambda i: (i, 0)),
    out_shape=jax.ShapeDtypeStruct((_N, _N_CLS), jnp.float32),
)


@jax.jit
def kernel(features, adj, W1_self, W1_neigh, b1, W2_self, W2_neigh, b2):
  src3d = adj[0].reshape(_NW, _NCHUNK, _CH)
  dst3d = adj[1].reshape(_NW, _NCHUNK, _CH)
  xext = jnp.concatenate(
      [features, jnp.ones((_N, _DX - _D_IN), jnp.float32)], axis=1)
  aggxp = _sc_agg_x(xext, src3d, dst3d).reshape(_NC, _N, _DX)
  hs, hw, rdeg = _tc1(features, aggxp, W1_self, W1_neigh,
                      b1.reshape(1, _D_HID), W2_self, W2_neigh)
  agg2p = _sc_agg_h(hw, src3d, dst3d).reshape(_NC, _N, _N_CLS)
  return _tc2(hs, agg2p, rdeg, b2.reshape(1, _N_CLS))


# speedup vs baseline: 10.2699x; 1.0230x over previous
import functools

import jax
import jax.numpy as jnp
from jax import lax
from jax.experimental import pallas as pl
from jax.experimental.pallas import tpu as pltpu
from jax.experimental.pallas import tpu_sc as plsc

_N = 10000
_E = 320000
_D_IN = 128
_D_HID = 256
_N_CLS = 64
_DX = _D_IN + 16

_NC = 2
_NS = 16
_NW = _NC * _NS
_EPT = _E // _NW
_CH = 100
_NCHUNK = _EPT // _CH
_NPAIR = _NCHUNK // 2
_RPT = _N // _NS


def _make_sc_agg(D):
  mesh = plsc.VectorSubcoreMesh(core_axis_name="c", subcore_axis_name="s")
  out_type = jax.ShapeDtypeStruct((_NC, _NS, _RPT, D), jnp.float32)
  scratch = [
      pltpu.VMEM((4, _CH), jnp.int32),
      pltpu.VMEM((4, _CH), jnp.int32),
      pltpu.VMEM((2, _CH, D), jnp.float32),
      pltpu.VMEM_SHARED((_N, D), jnp.float32),
      pltpu.SemaphoreType.DMA,
      pltpu.SemaphoreType.DMA,
      pltpu.SemaphoreType.DMA,
      pltpu.SemaphoreType.DMA,
      pltpu.SemaphoreType.DMA,
      pltpu.SemaphoreType.DMA,
  ]

  def body(table, src_hbm, dst_hbm, agg_out,
           srcv, dstv, rows, acc, isem0, isem1, gsem0, gsem1, ssem0, ssem1):
    cc = lax.axis_index("c")
    s = lax.axis_index("s")
    wid = s * _NC + cc

    def islot(c):
      return lax.rem(c, 4)

    def i_issue(c, sem):
      pltpu.async_copy(src_hbm.at[wid, c], srcv.at[islot(c)], sem)
      pltpu.async_copy(dst_hbm.at[wid, c], dstv.at[islot(c)], sem)

    def i_wait(c, sem):
      pltpu.make_async_copy(src_hbm.at[wid, c], srcv.at[islot(c)], sem).wait()
      pltpu.make_async_copy(dst_hbm.at[wid, c], dstv.at[islot(c)], sem).wait()

    def g_issue(c, b, sem):
      pltpu.async_copy(table.at[srcv.at[islot(c)]], rows.at[b], sem)

    def g_wait(c, b, sem):
      pltpu.make_async_copy(table.at[srcv.at[islot(c)]], rows.at[b],
                            sem).wait()

    def s_issue(c, b, sem):
      pltpu.async_copy(rows.at[b], acc.at[dstv.at[islot(c)]], sem, add=True)

    def s_wait(c, b, sem):
      pltpu.make_async_copy(rows.at[b], acc.at[dstv.at[islot(c)]],
                            sem).wait()

    i_issue(0, isem0)
    i_issue(1, isem1)

    zv = jnp.zeros((16,), jnp.float32)

    def zrow(i, _):
      for j in range(D // 16):
        rows[0, i, pl.ds(j * 16, 16)] = zv
      return 0

    lax.fori_loop(0, _CH, zrow, 0)
    for k in range(_RPT // _CH):
      pltpu.sync_copy(rows.at[0], acc.at[pl.ds(s * _RPT + k * _CH, _CH)])
    rem = _RPT - (_RPT // _CH) * _CH
    if rem:
      pltpu.sync_copy(rows.at[0, pl.ds(0, rem)],
                      acc.at[pl.ds(s * _RPT + (_RPT // _CH) * _CH, rem)])
    plsc.subcore_barrier()

    i_wait(0, isem0)
    i_issue(2, isem0)
    g_issue(0, 0, gsem0)

    def pair(p, _):
      ce = 2 * p
      co = 2 * p + 1
      g_wait(ce, 0, gsem0)
      s_issue(ce, 0, ssem0)

      @pl.when(p > 0)
      def _():
        s_wait(ce - 1, 1, ssem1)

      i_wait(ce + 1, isem1)

      @pl.when(ce + 3 < _NCHUNK)
      def _():
        i_issue(ce + 3, isem1)

      g_issue(ce + 1, 1, gsem1)
      g_wait(co, 1, gsem1)
      s_issue(co, 1, ssem1)
      s_wait(co - 1, 0, ssem0)

      @pl.when(co + 1 < _NCHUNK)
      def _():
        i_wait(co + 1, isem0)

      @pl.when(co + 3 < _NCHUNK)
      def _():
        i_issue(co + 3, isem0)

      @pl.when(co + 1 < _NCHUNK)
      def _():
        g_issue(co + 1, 0, gsem0)

      return 0

    lax.fori_loop(0, _NPAIR, pair, 0)
    s_wait(_NCHUNK - 1, 1, ssem1)
    plsc.subcore_barrier()

    pltpu.sync_copy(acc.at[pl.ds(s * _RPT, _RPT)], agg_out.at[cc, s])

  return pl.kernel(
      body, out_type=out_type, mesh=mesh, scratch_types=scratch,
      compiler_params=pltpu.CompilerParams(use_tc_tiling_on_sc=False))


_sc_agg_x = _make_sc_agg(_DX)
_sc_agg_h = _make_sc_agg(_N_CLS)

_BR = 400


def _tc1_body(x_ref, aggxp_ref, w1s_ref, w1n_ref, b1_ref,
              w2s_ref, w2n_ref, hs_ref, hw_ref, rdeg_ref):
  ap = aggxp_ref[0] + aggxp_ref[1]
  agg = ap[:, :_D_IN]
  deg = jnp.max(ap[:, _D_IN:], axis=1, keepdims=True)
  rdeg = 1.0 / jnp.maximum(deg, 1.0)
  hn = agg * rdeg
  h = (jnp.dot(x_ref[...], w1s_ref[...], preferred_element_type=jnp.float32)
       + jnp.dot(hn, w1n_ref[...], preferred_element_type=jnp.float32)
       + b1_ref[...])
  h = jnp.maximum(h, 0.0)
  hs_ref[...] = jnp.dot(h, w2s_ref[...], preferred_element_type=jnp.float32)
  hw_ref[...] = jnp.dot(h, w2n_ref[...], preferred_element_type=jnp.float32)
  rdeg_ref[...] = jnp.broadcast_to(rdeg, (_BR, _N_CLS))


_tc1 = pl.pallas_call(
    _tc1_body,
    grid=(_N // _BR,),
    in_specs=[
        pl.BlockSpec((_BR, _D_IN), lambda i: (i, 0)),
        pl.BlockSpec((_NC, _BR, _DX), lambda i: (0, i, 0)),
        pl.BlockSpec((_D_IN, _D_HID), lambda i: (0, 0)),
        pl.BlockSpec((_D_IN, _D_HID), lambda i: (0, 0)),
        pl.BlockSpec((1, _D_HID), lambda i: (0, 0)),
        pl.BlockSpec((_D_HID, _N_CLS), lambda i: (0, 0)),
        pl.BlockSpec((_D_HID, _N_CLS), lambda i: (0, 0)),
    ],
    out_specs=[
        pl.BlockSpec((_BR, _N_CLS), lambda i: (i, 0)),
        pl.BlockSpec((_BR, _N_CLS), lambda i: (i, 0)),
        pl.BlockSpec((_BR, _N_CLS), lambda i: (i, 0)),
    ],
    out_shape=[
        jax.ShapeDtypeStruct((_N, _N_CLS), jnp.float32),
        jax.ShapeDtypeStruct((_N, _N_CLS), jnp.float32),
        jax.ShapeDtypeStruct((_N, _N_CLS), jnp.float32),
    ],
)


def _tc2_body(hs_ref, agg2p_ref, rdeg_ref, b2_ref, out_ref):
  agg = agg2p_ref[0] + agg2p_ref[1]
  logits = hs_ref[...] + agg * rdeg_ref[...] + b2_ref[...]
  m = jnp.max(logits, axis=1, keepdims=True)
  lse = jnp.log(jnp.sum(jnp.exp(logits - m), axis=1, keepdims=True)) + m
  out_ref[...] = logits - lse


_tc2 = pl.pallas_call(
    _tc2_body,
    grid=(_N // _BR,),
    in_specs=[
        pl.BlockSpec((_BR, _N_CLS), lambda i: (i, 0)),
        pl.BlockSpec((_NC, _BR, _N_CLS), lambda i: (0, i, 0)),
        pl.BlockSpec((_BR, _N_CLS), lambda i: (i, 0)),
        pl.BlockSpec((1, _N_CLS), lambda i: (0, 0)),
    ],
    out_specs=pl.BlockSpec((_BR, _N_CLS), lambda i: (i, 0)),
    out_shape=jax.ShapeDtypeStruct((_N, _N_CLS), jnp.float32),
)


@jax.jit
def kernel(features, adj, W1_self, W1_neigh, b1, W2_self, W2_neigh, b2):
  src3d = adj[0].reshape(_NW, _NCHUNK, _CH)
  dst3d = adj[1].reshape(_NW, _NCHUNK, _CH)
  xext = jnp.concatenate(
      [features, jnp.ones((_N, _DX - _D_IN), jnp.float32)], axis=1)
  aggxp = _sc_agg_x(xext, src3d, dst3d).reshape(_NC, _N, _DX)
  hs, hw, rdeg = _tc1(features, aggxp, W1_self, W1_neigh,
                      b1.reshape(1, _D_HID), W2_self, W2_neigh)
  agg2p = _sc_agg_h(hw, src3d, dst3d).reshape(_NC, _N, _N_CLS)
  return _tc2(hs, agg2p, rdeg, b2.reshape(1, _N_CLS))


# speedup vs baseline: 10.8877x; 1.0601x over previous
import functools

import jax
import jax.numpy as jnp
from jax import lax
from jax.experimental import pallas as pl
from jax.experimental.pallas import tpu as pltpu
from jax.experimental.pallas import tpu_sc as plsc

_N = 10000
_E = 320000
_D_IN = 128
_D_HID = 256
_N_CLS = 64
_DX = _D_IN + 16

_NC = 2
_NS = 16
_NW = _NC * _NS
_EPT = _E // _NW
_CH = 80
_NCHUNK = _EPT // _CH
_NPAIR = _NCHUNK // 2
_RPT = 624
_REXTRA = _N - _NS * _RPT


def _make_sc_agg(D):
  mesh = plsc.VectorSubcoreMesh(core_axis_name="c", subcore_axis_name="s")
  out_type = jax.ShapeDtypeStruct((_NC, _N, D), jnp.float32)
  scratch = [
      pltpu.VMEM((4, _CH), jnp.int32),
      pltpu.VMEM((4, _CH), jnp.int32),
      pltpu.VMEM((2, _CH, D), jnp.float32),
      pltpu.VMEM_SHARED((_N, D), jnp.float32),
      pltpu.SemaphoreType.DMA,
      pltpu.SemaphoreType.DMA,
      pltpu.SemaphoreType.DMA,
      pltpu.SemaphoreType.DMA,
      pltpu.SemaphoreType.DMA,
      pltpu.SemaphoreType.DMA,
  ]

  def body(table, adj_hbm, agg_out,
           srcv, dstv, rows, acc, isem0, isem1, gsem0, gsem1, ssem0, ssem1):
    cc = lax.axis_index("c")
    s = lax.axis_index("s")
    wid = s * _NC + cc
    ebase = wid * _EPT

    def islot(c):
      return lax.rem(c, 4)

    def i_issue(c, sem):
      pltpu.async_copy(adj_hbm.at[0, pl.ds(ebase + c * _CH, _CH)],
                       srcv.at[islot(c)], sem)
      pltpu.async_copy(adj_hbm.at[1, pl.ds(ebase + c * _CH, _CH)],
                       dstv.at[islot(c)], sem)

    def i_wait(c, sem):
      pltpu.make_async_copy(adj_hbm.at[0, pl.ds(ebase + c * _CH, _CH)],
                            srcv.at[islot(c)], sem).wait()
      pltpu.make_async_copy(adj_hbm.at[1, pl.ds(ebase + c * _CH, _CH)],
                            dstv.at[islot(c)], sem).wait()

    def g_issue(c, b, sem):
      pltpu.async_copy(table.at[srcv.at[islot(c)]], rows.at[b], sem)

    def g_wait(c, b, sem):
      pltpu.make_async_copy(table.at[srcv.at[islot(c)]], rows.at[b],
                            sem).wait()

    def s_issue(c, b, sem):
      pltpu.async_copy(rows.at[b], acc.at[dstv.at[islot(c)]], sem, add=True)

    def s_wait(c, b, sem):
      pltpu.make_async_copy(rows.at[b], acc.at[dstv.at[islot(c)]],
                            sem).wait()

    i_issue(0, isem0)
    i_issue(1, isem1)

    zv = jnp.zeros((16,), jnp.float32)

    def zrow(i, _):
      for j in range(D // 16):
        rows[0, i, pl.ds(j * 16, 16)] = zv
      return 0

    lax.fori_loop(0, _CH, zrow, 0)
    for k in range(_RPT // _CH):
      pltpu.sync_copy(rows.at[0], acc.at[pl.ds(s * _RPT + k * _CH, _CH)])
    rem = _RPT - (_RPT // _CH) * _CH
    if rem:
      pltpu.sync_copy(rows.at[0, pl.ds(0, rem)],
                      acc.at[pl.ds(s * _RPT + (_RPT // _CH) * _CH, rem)])

    @pl.when(s == _NS - 1)
    def _():
      pltpu.sync_copy(rows.at[0, pl.ds(0, _REXTRA)],
                      acc.at[pl.ds(_NS * _RPT, _REXTRA)])

    plsc.subcore_barrier()

    i_wait(0, isem0)
    i_issue(2, isem0)
    g_issue(0, 0, gsem0)

    def pair(p, _):
      ce = 2 * p
      co = 2 * p + 1
      g_wait(ce, 0, gsem0)
      s_issue(ce, 0, ssem0)

      @pl.when(p > 0)
      def _():
        s_wait(ce - 1, 1, ssem1)

      i_wait(ce + 1, isem1)

      @pl.when(ce + 3 < _NCHUNK)
      def _():
        i_issue(ce + 3, isem1)

      g_issue(ce + 1, 1, gsem1)
      g_wait(co, 1, gsem1)
      s_issue(co, 1, ssem1)
      s_wait(co - 1, 0, ssem0)

      @pl.when(co + 1 < _NCHUNK)
      def _():
        i_wait(co + 1, isem0)

      @pl.when(co + 3 < _NCHUNK)
      def _():
        i_issue(co + 3, isem0)

      @pl.when(co + 1 < _NCHUNK)
      def _():
        g_issue(co + 1, 0, gsem0)

      return 0

    lax.fori_loop(0, _NPAIR, pair, 0)

    ct = _NCHUNK - 1
    g_wait(ct, 0, gsem0)
    s_issue(ct, 0, ssem0)
    s_wait(ct - 1, 1, ssem1)
    s_wait(ct, 0, ssem0)
    plsc.subcore_barrier()

    pltpu.sync_copy(acc.at[pl.ds(s * _RPT, _RPT)],
                    agg_out.at[cc, pl.ds(s * _RPT, _RPT)])

    @pl.when(s == _NS - 1)
    def _():
      pltpu.sync_copy(acc.at[pl.ds(_NS * _RPT, _REXTRA)],
                      agg_out.at[cc, pl.ds(_NS * _RPT, _REXTRA)])

  return pl.kernel(
      body, out_type=out_type, mesh=mesh, scratch_types=scratch,
      compiler_params=pltpu.CompilerParams(use_tc_tiling_on_sc=False))


_sc_agg_x = _make_sc_agg(_DX)
_sc_agg_h = _make_sc_agg(_N_CLS)


_BR = 2000


def _tc1_body(x_ref, aggxp_ref, w1s_ref, w1n_ref, b1_ref,
              w2s_ref, w2n_ref, hs_ref, hw_ref, rdeg_ref):
  ap = aggxp_ref[0] + aggxp_ref[1]
  agg = ap[:, :_D_IN]
  deg = jnp.max(ap[:, _D_IN:], axis=1, keepdims=True)
  rdeg = 1.0 / jnp.maximum(deg, 1.0)
  hn = agg * rdeg
  h = (jnp.dot(x_ref[...], w1s_ref[...], preferred_element_type=jnp.float32)
       + jnp.dot(hn, w1n_ref[...], preferred_element_type=jnp.float32)
       + b1_ref[...])
  h = jnp.maximum(h, 0.0)
  hs_ref[...] = jnp.dot(h, w2s_ref[...], preferred_element_type=jnp.float32)
  hw_ref[...] = jnp.dot(h, w2n_ref[...], preferred_element_type=jnp.float32)
  rdeg_ref[...] = jnp.broadcast_to(rdeg, (_BR, _N_CLS))


_tc1 = pl.pallas_call(
    _tc1_body,
    grid=(_N // _BR,),
    in_specs=[
        pl.BlockSpec((_BR, _D_IN), lambda i: (i, 0)),
        pl.BlockSpec((_NC, _BR, _DX), lambda i: (0, i, 0)),
        pl.BlockSpec((_D_IN, _D_HID), lambda i: (0, 0)),
        pl.BlockSpec((_D_IN, _D_HID), lambda i: (0, 0)),
        pl.BlockSpec((1, _D_HID), lambda i: (0, 0)),
        pl.BlockSpec((_D_HID, _N_CLS), lambda i: (0, 0)),
        pl.BlockSpec((_D_HID, _N_CLS), lambda i: (0, 0)),
    ],
    out_specs=[
        pl.BlockSpec((_BR, _N_CLS), lambda i: (i, 0)),
        pl.BlockSpec((_BR, _N_CLS), lambda i: (i, 0)),
        pl.BlockSpec((_BR, _N_CLS), lambda i: (i, 0)),
    ],
    out_shape=[
        jax.ShapeDtypeStruct((_N, _N_CLS), jnp.float32),
        jax.ShapeDtypeStruct((_N, _N_CLS), jnp.float32),
        jax.ShapeDtypeStruct((_N, _N_CLS), jnp.float32),
    ],
)


def _tc2_body(hs_ref, agg2p_ref, rdeg_ref, b2_ref, out_ref):
  agg = agg2p_ref[0] + agg2p_ref[1]
  logits = hs_ref[...] + agg * rdeg_ref[...] + b2_ref[...]
  m = jnp.max(logits, axis=1, keepdims=True)
  lse = jnp.log(jnp.sum(jnp.exp(logits - m), axis=1, keepdims=True)) + m
  out_ref[...] = logits - lse


_tc2 = pl.pallas_call(
    _tc2_body,
    grid=(_N // _BR,),
    in_specs=[
        pl.BlockSpec((_BR, _N_CLS), lambda i: (i, 0)),
        pl.BlockSpec((_NC, _BR, _N_CLS), lambda i: (0, i, 0)),
        pl.BlockSpec((_BR, _N_CLS), lambda i: (i, 0)),
        pl.BlockSpec((1, _N_CLS), lambda i: (0, 0)),
    ],
    out_specs=pl.BlockSpec((_BR, _N_CLS), lambda i: (i, 0)),
    out_shape=jax.ShapeDtypeStruct((_N, _N_CLS), jnp.float32),
)


@jax.jit
def kernel(features, adj, W1_self, W1_neigh, b1, W2_self, W2_neigh, b2):
  xext = lax.pad(features, jnp.float32(1.0),
                 [(0, 0, 0), (0, _DX - _D_IN, 0)])
  aggxp = _sc_agg_x(xext, adj)
  hs, hw, rdeg = _tc1(features, aggxp, W1_self, W1_neigh,
                      b1.reshape(1, _D_HID), W2_self, W2_neigh)
  agg2p = _sc_agg_h(hw, adj)
  return _tc2(hs, agg2p, rdeg, b2.reshape(1, _N_CLS))


# speedup vs baseline: 12.6513x; 1.1620x over previous
import functools

import jax
import jax.numpy as jnp
from jax import lax
from jax.experimental import pallas as pl
from jax.experimental.pallas import tpu as pltpu
from jax.experimental.pallas import tpu_sc as plsc

_N = 10000
_E = 320000
_D_IN = 128
_D_HID = 256
_N_CLS = 64
_DX = _D_IN + 16

_NC = 2
_NS = 16
_NW = _NC * _NS
_EPT = _E // _NW
_CH = 128
_NCHUNK = _EPT // _CH
_NPAIR = _NCHUNK // 2
_TAIL = _EPT - _NCHUNK * _CH
_RPT = 624
_REXTRA = _N - _NS * _RPT


def _make_sc_agg(D):
  mesh = plsc.VectorSubcoreMesh(core_axis_name="c", subcore_axis_name="s")
  out_type = jax.ShapeDtypeStruct((_NC, _N, D), jnp.float32)
  scratch = [
      pltpu.VMEM((4, _CH), jnp.int32),
      pltpu.VMEM((4, _CH), jnp.int32),
      pltpu.VMEM((2, _CH, D), jnp.float32),
      pltpu.VMEM_SHARED((_N, D), jnp.float32),
      pltpu.SemaphoreType.DMA,
      pltpu.SemaphoreType.DMA,
      pltpu.SemaphoreType.DMA,
      pltpu.SemaphoreType.DMA,
      pltpu.SemaphoreType.DMA,
      pltpu.SemaphoreType.DMA,
  ]

  def body(table, adj_hbm, agg_out,
           srcv, dstv, rows, acc, isem0, isem1, gsem0, gsem1, ssem0, ssem1):
    cc = lax.axis_index("c")
    s = lax.axis_index("s")
    wid = s * _NC + cc
    ebase = wid * _EPT

    def islot(c):
      return lax.rem(c, 4)

    def i_issue(c, sem):
      pltpu.async_copy(adj_hbm.at[0, pl.ds(ebase + c * _CH, _CH)],
                       srcv.at[islot(c)], sem)
      pltpu.async_copy(adj_hbm.at[1, pl.ds(ebase + c * _CH, _CH)],
                       dstv.at[islot(c)], sem)

    def i_wait(c, sem):
      pltpu.make_async_copy(adj_hbm.at[0, pl.ds(ebase + c * _CH, _CH)],
                            srcv.at[islot(c)], sem).wait()
      pltpu.make_async_copy(adj_hbm.at[1, pl.ds(ebase + c * _CH, _CH)],
                            dstv.at[islot(c)], sem).wait()

    def g_issue(c, b, sem):
      pltpu.async_copy(table.at[srcv.at[islot(c)]], rows.at[b], sem)

    def g_wait(c, b, sem):
      pltpu.make_async_copy(table.at[srcv.at[islot(c)]], rows.at[b],
                            sem).wait()

    def s_issue(c, b, sem):
      pltpu.async_copy(rows.at[b], acc.at[dstv.at[islot(c)]], sem, add=True)

    def s_wait(c, b, sem):
      pltpu.make_async_copy(rows.at[b], acc.at[dstv.at[islot(c)]],
                            sem).wait()

    i_issue(0, isem0)
    i_issue(1, isem1)

    zv = jnp.zeros((16,), jnp.float32)

    def zrow(i, _):
      for j in range(D // 16):
        rows[0, i, pl.ds(j * 16, 16)] = zv
      return 0

    lax.fori_loop(0, _CH, zrow, 0)
    for k in range(_RPT // _CH):
      pltpu.sync_copy(rows.at[0], acc.at[pl.ds(s * _RPT + k * _CH, _CH)])
    rem = _RPT - (_RPT // _CH) * _CH
    if rem:
      pltpu.sync_copy(rows.at[0, pl.ds(0, rem)],
                      acc.at[pl.ds(s * _RPT + (_RPT // _CH) * _CH, rem)])

    @pl.when(s == _NS - 1)
    def _():
      pltpu.sync_copy(rows.at[0, pl.ds(0, _REXTRA)],
                      acc.at[pl.ds(_NS * _RPT, _REXTRA)])

    plsc.subcore_barrier()

    i_wait(0, isem0)
    i_issue(2, isem0)
    g_issue(0, 0, gsem0)

    def pair(p, _):
      ce = 2 * p
      co = 2 * p + 1
      g_wait(ce, 0, gsem0)
      s_issue(ce, 0, ssem0)

      @pl.when(p > 0)
      def _():
        s_wait(ce - 1, 1, ssem1)

      i_wait(ce + 1, isem1)

      @pl.when(ce + 3 < _NCHUNK)
      def _():
        i_issue(ce + 3, isem1)

      g_issue(ce + 1, 1, gsem1)
      g_wait(co, 1, gsem1)
      s_issue(co, 1, ssem1)
      s_wait(co - 1, 0, ssem0)

      @pl.when(co + 1 < _NCHUNK)
      def _():
        i_wait(co + 1, isem0)

      @pl.when(co + 3 < _NCHUNK)
      def _():
        i_issue(co + 3, isem0)

      @pl.when(co + 1 < _NCHUNK)
      def _():
        g_issue(co + 1, 0, gsem0)

      return 0

    lax.fori_loop(0, _NPAIR, pair, 0)

    tbase = ebase + _NCHUNK * _CH
    pltpu.async_copy(adj_hbm.at[0, pl.ds(tbase, _TAIL)],
                     srcv.at[0, pl.ds(0, _TAIL)], isem0)
    pltpu.async_copy(adj_hbm.at[1, pl.ds(tbase, _TAIL)],
                     dstv.at[0, pl.ds(0, _TAIL)], isem0)
    s_wait(_NCHUNK - 1, 1, ssem1)
    pltpu.make_async_copy(adj_hbm.at[0, pl.ds(tbase, _TAIL)],
                          srcv.at[0, pl.ds(0, _TAIL)], isem0).wait()
    pltpu.make_async_copy(adj_hbm.at[1, pl.ds(tbase, _TAIL)],
                          dstv.at[0, pl.ds(0, _TAIL)], isem0).wait()
    pltpu.async_copy(table.at[srcv.at[0, pl.ds(0, _TAIL)]],
                     rows.at[0, pl.ds(0, _TAIL)], gsem0)
    pltpu.make_async_copy(table.at[srcv.at[0, pl.ds(0, _TAIL)]],
                          rows.at[0, pl.ds(0, _TAIL)], gsem0).wait()
    pltpu.async_copy(rows.at[0, pl.ds(0, _TAIL)],
                     acc.at[dstv.at[0, pl.ds(0, _TAIL)]], ssem0, add=True)
    pltpu.make_async_copy(rows.at[0, pl.ds(0, _TAIL)],
                          acc.at[dstv.at[0, pl.ds(0, _TAIL)]], ssem0).wait()
    plsc.subcore_barrier()

    pltpu.sync_copy(acc.at[pl.ds(s * _RPT, _RPT)],
                    agg_out.at[cc, pl.ds(s * _RPT, _RPT)])

    @pl.when(s == _NS - 1)
    def _():
      pltpu.sync_copy(acc.at[pl.ds(_NS * _RPT, _REXTRA)],
                      agg_out.at[cc, pl.ds(_NS * _RPT, _REXTRA)])

  return pl.kernel(
      body, out_type=out_type, mesh=mesh, scratch_types=scratch,
      compiler_params=pltpu.CompilerParams(use_tc_tiling_on_sc=False))


_sc_agg_x = _make_sc_agg(_DX)
_sc_agg_h = _make_sc_agg(_N_CLS)


_BR = 2000


def _tc1_body(x_ref, aggxp_ref, w1s_ref, w1n_ref, b1_ref,
              w2s_ref, w2n_ref, hs_ref, hw_ref, rdeg_ref):
  ap = aggxp_ref[0] + aggxp_ref[1]
  agg = ap[:, :_D_IN]
  deg = jnp.max(ap[:, _D_IN:], axis=1, keepdims=True)
  rdeg = 1.0 / jnp.maximum(deg, 1.0)
  hn = agg * rdeg
  h = (jnp.dot(x_ref[...], w1s_ref[...], preferred_element_type=jnp.float32)
       + jnp.dot(hn, w1n_ref[...], preferred_element_type=jnp.float32)
       + b1_ref[...])
  h = jnp.maximum(h, 0.0)
  hs_ref[...] = jnp.dot(h, w2s_ref[...], preferred_element_type=jnp.float32)
  hw_ref[...] = jnp.dot(h, w2n_ref[...], preferred_element_type=jnp.float32)
  rdeg_ref[...] = jnp.broadcast_to(rdeg, (_BR, _N_CLS))


_tc1 = pl.pallas_call(
    _tc1_body,
    grid=(_N // _BR,),
    in_specs=[
        pl.BlockSpec((_BR, _D_IN), lambda i: (i, 0)),
        pl.BlockSpec((_NC, _BR, _DX), lambda i: (0, i, 0)),
        pl.BlockSpec((_D_IN, _D_HID), lambda i: (0, 0)),
        pl.BlockSpec((_D_IN, _D_HID), lambda i: (0, 0)),
        pl.BlockSpec((1, _D_HID), lambda i: (0, 0)),
        pl.BlockSpec((_D_HID, _N_CLS), lambda i: (0, 0)),
        pl.BlockSpec((_D_HID, _N_CLS), lambda i: (0, 0)),
    ],
    out_specs=[
        pl.BlockSpec((_BR, _N_CLS), lambda i: (i, 0)),
        pl.BlockSpec((_BR, _N_CLS), lambda i: (i, 0)),
        pl.BlockSpec((_BR, _N_CLS), lambda i: (i, 0)),
    ],
    out_shape=[
        jax.ShapeDtypeStruct((_N, _N_CLS), jnp.float32),
        jax.ShapeDtypeStruct((_N, _N_CLS), jnp.float32),
        jax.ShapeDtypeStruct((_N, _N_CLS), jnp.float32),
    ],
)


def _tc2_body(hs_ref, agg2p_ref, rdeg_ref, b2_ref, out_ref):
  agg = agg2p_ref[0] + agg2p_ref[1]
  logits = hs_ref[...] + agg * rdeg_ref[...] + b2_ref[...]
  m = jnp.max(logits, axis=1, keepdims=True)
  lse = jnp.log(jnp.sum(jnp.exp(logits - m), axis=1, keepdims=True)) + m
  out_ref[...] = logits - lse


_tc2 = pl.pallas_call(
    _tc2_body,
    grid=(_N // _BR,),
    in_specs=[
        pl.BlockSpec((_BR, _N_CLS), lambda i: (i, 0)),
        pl.BlockSpec((_NC, _BR, _N_CLS), lambda i: (0, i, 0)),
        pl.BlockSpec((_BR, _N_CLS), lambda i: (i, 0)),
        pl.BlockSpec((1, _N_CLS), lambda i: (0, 0)),
    ],
    out_specs=pl.BlockSpec((_BR, _N_CLS), lambda i: (i, 0)),
    out_shape=jax.ShapeDtypeStruct((_N, _N_CLS), jnp.float32),
)


@jax.jit
def kernel(features, adj, W1_self, W1_neigh, b1, W2_self, W2_neigh, b2):
  xext = lax.pad(features, jnp.float32(1.0),
                 [(0, 0, 0), (0, _DX - _D_IN, 0)])
  aggxp = _sc_agg_x(xext, adj)
  hs, hw, rdeg = _tc1(features, aggxp, W1_self, W1_neigh,
                      b1.reshape(1, _D_HID), W2_self, W2_neigh)
  agg2p = _sc_agg_h(hw, adj)
  return _tc2(hs, agg2p, rdeg, b2.reshape(1, _N_CLS))
